# baseline (speedup 1.0000x reference)
import jax
import jax.numpy as jnp
from jax.experimental import pallas as pl
from jax.experimental.pallas import tpu as pltpu

BATCH = 2048
N_ENT = 1024
D_DIM = 32
P_SENT = 1024
ROWS = 4


def _update_body(starts_ref, counts_ref, perm_ref,
                 e_ref, u_ref, v_ref, w_ref, hid_ref, key_ref, out_ref):
    i = pl.program_id(0)
    U = u_ref[...]
    V = v_ref[...]
    W = w_ref[...]
    for r in range(ROWS):
        b = i * ROWS + r
        h = hid_ref[r]
        s0 = starts_ref[b]
        cnt = counts_ref[b]

        def hit_fn():
            k = key_ref[r]
            base = (jnp.dot(h, U, preferred_element_type=jnp.float32)
                    + jnp.dot(k, V, preferred_element_type=jnp.float32))
            s = h + k

            def loop(j, acc):
                p = perm_ref[j]
                e = e_ref[pl.ds(p, 1), :]
                gate = jax.nn.sigmoid(
                    jnp.sum(s * e, axis=1, keepdims=True))
                ew = jnp.dot(e, W, preferred_element_type=jnp.float32)
                ht = jnp.maximum(base + ew, 0.0)
                return acc + gate * ht

            acc = jax.lax.fori_loop(
                s0, s0 + cnt, loop, jnp.zeros((N_ENT, D_DIM), jnp.float32))
            return h + acc

        hnew = jax.lax.cond(cnt > 0, hit_fn, lambda: h)
        ss = jnp.sum(hnew * hnew, axis=1, keepdims=True)
        out_ref[r] = hnew * jax.lax.rsqrt(jnp.maximum(ss, 1e-12))


def _csr_route(indices):
    counts = jnp.zeros((BATCH,), jnp.int32).at[indices].add(1)
    starts = jnp.concatenate(
        [jnp.zeros((1,), jnp.int32), jnp.cumsum(counts)[:-1].astype(jnp.int32)])
    perm = jnp.argsort(indices).astype(jnp.int32)
    return starts, counts, perm


def kernel(encoded_sents, indices, hiddens, keys, U, V, W):
    starts, counts, perm = _csr_route(indices)
    grid_spec = pltpu.PrefetchScalarGridSpec(
        num_scalar_prefetch=3,
        grid=(BATCH // ROWS,),
        in_specs=[
            pl.BlockSpec((P_SENT, D_DIM), lambda i, *_: (0, 0)),
            pl.BlockSpec((D_DIM, D_DIM), lambda i, *_: (0, 0)),
            pl.BlockSpec((D_DIM, D_DIM), lambda i, *_: (0, 0)),
            pl.BlockSpec((D_DIM, D_DIM), lambda i, *_: (0, 0)),
            pl.BlockSpec((ROWS, N_ENT, D_DIM), lambda i, *_: (i, 0, 0)),
            pl.BlockSpec((ROWS, N_ENT, D_DIM), lambda i, *_: (i, 0, 0)),
        ],
        out_specs=pl.BlockSpec((ROWS, N_ENT, D_DIM), lambda i, *_: (i, 0, 0)),
    )
    return pl.pallas_call(
        _update_body,
        grid_spec=grid_spec,
        out_shape=jax.ShapeDtypeStruct((BATCH, N_ENT, D_DIM), jnp.float32),
        compiler_params=pltpu.CompilerParams(
            dimension_semantics=("arbitrary",)),
    )(starts, counts, perm, encoded_sents, U, V, W, hiddens, keys)


# speedup vs baseline: 1.5204x; 1.5204x over previous
import jax
import jax.numpy as jnp
from jax.experimental import pallas as pl
from jax.experimental.pallas import tpu as pltpu

BATCH = 2048
N_ENT = 1024
D_DIM = 32
P_SENT = 1024
G = 4
NR = N_ENT // G
LN = G * D_DIM
RB = 8
_EPS = 1e-12


def _norm_body(h_ref, e_ref, w4_ref, seg_ref, out_ref, ew_ref):
    @pl.when(pl.program_id(0) == 0)
    def _():
        ew_ref[...] = jnp.dot(e_ref[...], w4_ref[...],
                              preferred_element_type=jnp.float32)

    seg = seg_ref[...]
    for r in range(RB):
        x = h_ref[r]
        ss = jnp.dot(x * x, seg, preferred_element_type=jnp.float32)
        out_ref[r] = x * jax.lax.rsqrt(jnp.maximum(ss, _EPS))


def _update_body(sidx_ref, perm_ref, e_ref, ew_ref, u_ref, v_ref, seg_ref,
                 h_ref, k_ref, bout_ref, out_ref):
    del perm_ref, bout_ref
    j = pl.program_id(0)
    b = sidx_ref[j]
    jm1 = jnp.maximum(j - 1, 0)
    is_new = jnp.logical_or(j == 0, sidx_ref[jm1] != b)
    is_last = sidx_ref[j + 1] != b

    h = h_ref[0]
    k = k_ref[0]
    e = e_ref[0]
    ew = ew_ref[0]
    seg = seg_ref[...]

    gate = jax.nn.sigmoid(
        jnp.dot((h + k) * e, seg, preferred_element_type=jnp.float32))
    cand = jnp.maximum(
        jnp.dot(h, u_ref[...], preferred_element_type=jnp.float32)
        + jnp.dot(k, v_ref[...], preferred_element_type=jnp.float32)
        + ew, 0.0)
    upd = gate * cand

    @pl.when(is_new)
    def _():
        out_ref[0] = h + upd

    @pl.when(jnp.logical_not(is_new))
    def _():
        out_ref[0] = out_ref[0] + upd

    @pl.when(is_last)
    def _():
        x = out_ref[0]
        ss = jnp.dot(x * x, seg, preferred_element_type=jnp.float32)
        out_ref[0] = x * jax.lax.rsqrt(jnp.maximum(ss, _EPS))


def kernel(encoded_sents, indices, hiddens, keys, U, V, W):
    f32 = jnp.float32
    H4 = hiddens.reshape(BATCH, NR, LN)
    K4 = keys.reshape(BATCH, NR, LN)
    E4 = jnp.tile(encoded_sents, (1, G))
    eye = jnp.eye(G, dtype=f32)
    U4 = jnp.kron(eye, U)
    V4 = jnp.kron(eye, V)
    W4 = jnp.kron(eye, W)
    SEG = jnp.kron(eye, jnp.ones((D_DIM, D_DIM), f32))

    perm = jnp.argsort(indices).astype(jnp.int32)
    sidx = jnp.take(indices, perm)
    sidx_pad = jnp.concatenate([sidx, jnp.full((1,), -1, jnp.int32)])

    bout, ew_tab = pl.pallas_call(
        _norm_body,
        grid=(BATCH // RB,),
        in_specs=[
            pl.BlockSpec((RB, NR, LN), lambda i: (i, 0, 0)),
            pl.BlockSpec((P_SENT, LN), lambda i: (0, 0)),
            pl.BlockSpec((LN, LN), lambda i: (0, 0)),
            pl.BlockSpec((LN, LN), lambda i: (0, 0)),
        ],
        out_specs=[
            pl.BlockSpec((RB, NR, LN), lambda i: (i, 0, 0)),
            pl.BlockSpec((P_SENT, LN), lambda i: (0, 0)),
        ],
        out_shape=[
            jax.ShapeDtypeStruct((BATCH, NR, LN), f32),
            jax.ShapeDtypeStruct((P_SENT, LN), f32),
        ],
        compiler_params=pltpu.CompilerParams(
            dimension_semantics=("arbitrary",)),
    )(H4, E4, W4, SEG)

    E4r = E4.reshape(P_SENT, 1, LN)
    EWr = ew_tab.reshape(P_SENT, 1, LN)

    grid_spec = pltpu.PrefetchScalarGridSpec(
        num_scalar_prefetch=2,
        grid=(P_SENT,),
        in_specs=[
            pl.BlockSpec((1, 1, LN), lambda j, sidx, perm: (perm[j], 0, 0)),
            pl.BlockSpec((1, 1, LN), lambda j, sidx, perm: (perm[j], 0, 0)),
            pl.BlockSpec((LN, LN), lambda j, *_: (0, 0)),
            pl.BlockSpec((LN, LN), lambda j, *_: (0, 0)),
            pl.BlockSpec((LN, LN), lambda j, *_: (0, 0)),
            pl.BlockSpec((1, NR, LN), lambda j, sidx, perm: (sidx[j], 0, 0)),
            pl.BlockSpec((1, NR, LN), lambda j, sidx, perm: (sidx[j], 0, 0)),
            pl.BlockSpec(memory_space=pl.ANY),
        ],
        out_specs=pl.BlockSpec((1, NR, LN), lambda j, sidx, perm: (sidx[j], 0, 0)),
    )
    out = pl.pallas_call(
        _update_body,
        grid_spec=grid_spec,
        out_shape=jax.ShapeDtypeStruct((BATCH, NR, LN), f32),
        input_output_aliases={9: 0},
        compiler_params=pltpu.CompilerParams(
            dimension_semantics=("arbitrary",)),
    )(sidx_pad, perm, E4r, EWr, U4, V4, SEG, H4, K4, bout)

    return out.reshape(BATCH, N_ENT, D_DIM)
